# baseline (speedup 1.0000x reference)
import jax
import jax.numpy as jnp
from jax import lax
from jax.experimental import pallas as pl
from jax.experimental.pallas import tpu as pltpu
from jax.experimental.pallas import tpu_sc as plsc

B = 64
Q = 900
C = 91
K = 300
QC = Q * C
L = 16
NV = (QC + L - 1) // L
QCP = NV * L
NC, NS = 2, 16
NW = NC * NS
BPW = B // NW
SEL = 320
KP = 304
HB = 1024
RS = HB * L
CAP = 2560
IDX_PAD = 0x7FFFFF00


def _take(v, idx):
    return lax.gather(
        v, idx[:, None],
        lax.GatherDimensionNumbers(
            offset_dims=(), collapsed_slice_dims=(0,), start_index_map=(0,)),
        (1,), mode=lax.GatherScatterMode.PROMISE_IN_BOUNDS)


def _splat(x, dtype=jnp.int32):
    return jnp.broadcast_to(jnp.asarray(x, dtype), (L,))


def _sc_body(prob_hbm, boxes_hbm, scale_hbm,
             scores_hbm, labels_hbm, boxes_out_hbm,
             p_v, boxes_v, scale_v, hist, hist2, sel_val, sel_idx,
             cand_val, cand_idx, oscore, olabel, oboxid, oboxes):
    cid = lax.axis_index("c")
    sid = lax.axis_index("s")
    wid = sid * NC + cid

    lanes = lax.iota(jnp.int32, L)
    ones = jnp.ones((L,), jnp.int32)
    lane_h = lanes * HB
    lane_r = lanes * 32
    rep4 = lanes >> 2
    mod4 = lanes & 3
    idx_cxy = (rep4 << 2) + (lanes & 1)
    idx_wh = idx_cxy + 2
    coef = jnp.where((lanes & 2) == 0, -0.5, 0.5).astype(jnp.float32)

    shift = [jnp.int32(0)]

    def load_p(i):
        v = p_v[pl.ds(i * L + shift[0], L)]
        u = lax.bitcast_convert_type(v, jnp.int32)
        return v, u

    def in_group_pick(acc, needg):
        rev = lax.rev(acc, (0,))
        cs = plsc.cumsum(rev)
        m = cs >= _splat(needg)
        k = jnp.broadcast_to(plsc.all_reduce_ffs(m), (L,))
        sel = lanes == k
        k_s = jnp.sum(jnp.where(sel, lanes, 0))
        abv = jnp.sum(jnp.where(sel, cs - rev, 0))
        return k_s, abv

    def scan1024(need):
        @plsc.parallel_loop(0, HB, unroll=8)
        def _(t):
            hist[pl.ds(t * L, L)] = (hist[pl.ds(t * L, L)]
                                     + hist2[pl.ds(t * L, L)])

        @plsc.parallel_loop(
            0, 64, unroll=1,
            carry=(jnp.int32(0), jnp.int32(0), jnp.int32(0),
                   jnp.bool_(False)))
        def g_scan(t, carry):
            cum, gsel, above, found = carry
            g = 63 - t
            acc = hist[pl.ds(16 * g, L)]
            for l in range(1, L):
                acc = acc + hist[pl.ds(l * HB + 16 * g, L)]
            hist2[pl.ds(16 * g, L)] = acc
            tg = jnp.sum(acc)
            hit = jnp.logical_and(jnp.logical_not(found), cum + tg >= need)
            gsel = jnp.where(hit, g, gsel)
            above = jnp.where(hit, cum, above)
            found = jnp.logical_or(found, hit)
            return cum + tg, gsel, above, found
        _, gsel, above, _ = g_scan
        acc = hist2[pl.ds(16 * gsel, L)]
        k_s, abv_g = in_group_pick(acc, need - above)
        bucket = 16 * gsel + 15 - k_s
        above = above + abv_g
        cnt = jnp.sum(jnp.where(lanes == jnp.broadcast_to(k_s, (L,)),
                                lax.rev(acc, (0,)), 0))
        return bucket, above, cnt

    def scan32(need):
        t0 = hist[pl.ds(0, L)]
        t1 = hist[pl.ds(16, L)]
        for l in range(1, L):
            t0 = t0 + hist[pl.ds(l * 32, L)]
            t1 = t1 + hist[pl.ds(l * 32 + 16, L)]
        c1 = jnp.sum(t1)
        in_hi = need <= c1
        acc = jnp.where(_splat(in_hi, jnp.bool_), t1, t0)
        needg = jnp.where(in_hi, need, need - c1)
        k_s, abv_g = in_group_pick(acc, needg)
        bucket = jnp.where(in_hi, 16, 0) + 15 - k_s
        above = jnp.where(in_hi, jnp.int32(0), c1) + abv_g
        return bucket, above

    def zero_hist(n_vregs, both=False):
        @plsc.parallel_loop(0, n_vregs, unroll=8)
        def _(t):
            hist[pl.ds(t * L, L)] = jnp.zeros((L,), jnp.int32)
            if both:
                hist2[pl.ds(t * L, L)] = jnp.zeros((L,), jnp.int32)

    def append(buf_v, buf_i, off, m, v, gidx):
        mi = m.astype(jnp.int32)
        pos = off + plsc.cumsum(mi) - mi
        plsc.store_scatter(buf_v, [pos], v, mask=m)
        plsc.store_scatter(buf_i, [pos], gidx, mask=m)
        return off + plsc.all_reduce_population_count(m)

    def process(b):
        r = 4 * (b & 1)
        shift[0] = r
        s0 = pl.multiple_of(b * QC - r, 8)
        pltpu.sync_copy(prob_hbm.at[pl.ds(s0, QCP)], p_v.at[pl.ds(0, QCP)])
        pltpu.sync_copy(boxes_hbm.at[b], boxes_v)
        pltpu.sync_copy(scale_hbm.at[b], scale_v)

        tail = p_v[pl.ds(QCP - L + r, L)]
        p_v[pl.ds(QCP - L + r, L)] = jnp.where(lanes < (L - (QCP - QC)),
                                               tail, 0.0)

        zero_hist(HB, both=True)

        def hist1_one(i, hb):
            v, u = load_p(i)
            d = lax.shift_right_logical(u, 20)
            plsc.addupdate_scatter(hb, [lane_h + d], ones)

        def hist1(g, _):
            for t in range(4):
                hist1_one(g * 4 + t, hist if t % 2 == 0 else hist2)
            return 0
        lax.fori_loop(0, NV // 4, hist1, 0)
        for i in range(NV - NV % 4, NV):
            hist1_one(jnp.int32(i), hist if i % 2 == 0 else hist2)
        b1, a1, cnt_b1 = scan1024(jnp.int32(K))
        need1 = jnp.int32(K) - a1

        @plsc.parallel_loop(0, SEL // L, unroll=4)
        def _(j):
            sel_val[pl.ds(j * L, L)] = jnp.full((L,), -1.0, jnp.float32)
            sel_idx[pl.ds(j * L, L)] = jnp.full((L,), IDX_PAD, jnp.int32)

        @plsc.parallel_loop(0, KP // L, unroll=4)
        def _(j):
            oboxid[pl.ds(j * L, L)] = jnp.zeros((L,), jnp.int32)

        def rounds_5bit(load_fn, nv, pfx0, need0):
            pfx, nd = pfx0, need0
            for shift in (15, 10, 5, 0):
                zero_hist(32)

                def hbody(i, _, shift=shift, pfx=pfx):
                    v, u, valid = load_fn(i)
                    m = lax.shift_right_logical(u, shift + 5) == _splat(pfx)
                    if valid is not None:
                        m = jnp.logical_and(valid, m)
                    d = lax.shift_right_logical(u, shift) & 31
                    plsc.addupdate_scatter(hist, [lane_r + d], ones, mask=m)
                    return 0
                lax.fori_loop(0, nv, hbody, 0)
                dsel, abv = scan32(nd)
                pfx = (pfx << 5) | dsel
                nd = nd - abv
            return pfx, nd

        def fast_path():
            @plsc.parallel_loop(0, NV, unroll=4, carry=(_splat(0), _splat(0)))
            def pass2(i, carry):
                off_s, off_c = carry
                v, u = load_p(i)
                gidx = _splat(i * L) + lanes
                d = lax.shift_right_logical(u, 20)
                off_s = append(sel_val, sel_idx, off_s, d > b1, v, gidx)
                off_c = append(cand_val, cand_idx, off_c, d == b1, v, gidx)
                return off_s, off_c

            ncv = lax.div(cnt_b1 + (L - 1), jnp.int32(L))

            def load_c(i):
                v = cand_val[pl.ds(i * L, L)]
                u = lax.bitcast_convert_type(v, jnp.int32)
                valid = (_splat(i * L) + lanes) < _splat(cnt_b1)
                return v, u, valid

            t_bits, need = rounds_5bit(load_c, ncv, b1, need1)

            @plsc.parallel_loop(0, ncv, unroll=4,
                                carry=(_splat(a1), _splat(0)))
            def _(i, carry):
                off, taken = carry
                v, u, valid = load_c(i)
                gidx = cand_idx[pl.ds(i * L, L)]
                m_gt = jnp.logical_and(u > t_bits, valid)
                m_eq = jnp.logical_and(
                    jnp.logical_and(u == t_bits, valid),
                    taken < _splat(need))
                off = append(sel_val, sel_idx,
                             off, jnp.logical_or(m_gt, m_eq), v, gidx)
                taken = taken + plsc.all_reduce_population_count(m_eq)
                return off, taken
            return jnp.int32(0)

        def slow_path():
            def load_f(i):
                v, u = load_p(i)
                return v, u, None

            t_bits, need = rounds_5bit(load_f, jnp.int32(NV), b1, need1)

            @plsc.parallel_loop(0, NV, unroll=4,
                                carry=(_splat(0), _splat(0)))
            def _(i, carry):
                off, taken = carry
                v, u = load_p(i)
                gidx = _splat(i * L) + lanes
                m_gt = u > t_bits
                m_eq = jnp.logical_and(u == t_bits, taken < _splat(need))
                off = append(sel_val, sel_idx,
                             off, jnp.logical_or(m_gt, m_eq), v, gidx)
                taken = taken + plsc.all_reduce_population_count(m_eq)
                return off, taken
            return jnp.int32(0)

        lax.cond(cnt_b1 <= CAP, fast_path, slow_path)

        @plsc.parallel_loop(0, SEL, unroll=4)
        def _(i):
            base = i - (i & (L - 1))
            lane = _splat(i & (L - 1))
            vi = _take(sel_val[pl.ds(base, L)], lane)
            ii = _take(sel_idx[pl.ds(base, L)], lane)
            i_s = _splat(i)
            rank = jnp.zeros((L,), jnp.int32)
            for j in range(SEL // L):
                va = sel_val[pl.ds(j * L, L)]
                pos = _splat(j * L) + lanes
                m = jnp.logical_or(
                    va > vi,
                    jnp.logical_and(va == vi, pos < i_s))
                rank = rank + plsc.all_reduce_population_count(m)
            mw = jnp.logical_and(lanes == 0, rank < K)
            plsc.store_scatter(oscore, [rank], vi, mask=mw)
            plsc.store_scatter(olabel, [rank], lax.rem(ii, _splat(C)),
                               mask=mw)
            plsc.store_scatter(oboxid, [rank], lax.div(ii, _splat(C)),
                               mask=mw)

        scale = scale_v[:]
        for vv in range(KP // 4):
            base = (4 * vv) & ~(L - 1)
            rows = _take(oboxid[pl.ds(base, L)], _splat(4 * vv - base) + rep4)
            bx = plsc.load_gather(boxes_v, [rows * 4 + mod4])
            out = (_take(bx, idx_cxy) + coef * _take(bx, idx_wh)) * scale
            oboxes[pl.ds(16 * vv, L)] = out

        pltpu.sync_copy(oscore, scores_hbm.at[b])
        pltpu.sync_copy(olabel, labels_hbm.at[b])
        pltpu.sync_copy(oboxes, boxes_out_hbm.at[b])

    for k in range(BPW):
        process(wid + NW * k)


@jax.jit
def _post_process_sc(prob, boxes_flat, scale):
    mesh = plsc.VectorSubcoreMesh(core_axis_name="c", subcore_axis_name="s",
                                  num_cores=NC, num_subcores=NS)
    fn = pl.kernel(
        _sc_body,
        out_type=[
            jax.ShapeDtypeStruct((B, KP), jnp.float32),
            jax.ShapeDtypeStruct((B, KP), jnp.int32),
            jax.ShapeDtypeStruct((B, 4 * KP), jnp.float32),
        ],
        mesh=mesh,
        compiler_params=pltpu.CompilerParams(needs_layout_passes=False,
                                             use_tc_tiling_on_sc=True),
        scratch_types=[
            pltpu.VMEM((QCP + L,), jnp.float32),
            pltpu.VMEM((4 * Q,), jnp.float32),
            pltpu.VMEM((L,), jnp.float32),
            pltpu.VMEM((RS,), jnp.int32),
            pltpu.VMEM((RS,), jnp.int32),
            pltpu.VMEM((SEL,), jnp.float32),
            pltpu.VMEM((SEL,), jnp.int32),
            pltpu.VMEM((CAP + L,), jnp.float32),
            pltpu.VMEM((CAP + L,), jnp.int32),
            pltpu.VMEM((KP,), jnp.float32),
            pltpu.VMEM((KP,), jnp.int32),
            pltpu.VMEM((KP,), jnp.int32),
            pltpu.VMEM((4 * KP,), jnp.float32),
        ],
    )
    return fn(prob, boxes_flat, scale)


def kernel(outputs_pred_logits, outputs_pred_boxes, target_sizes, image_names):
    prob = jax.nn.sigmoid(outputs_pred_logits).reshape(B * QC)
    boxes_flat = outputs_pred_boxes.reshape(B, 4 * Q)
    img_h = target_sizes[:, 0].astype(jnp.float32)
    img_w = target_sizes[:, 1].astype(jnp.float32)
    scale = jnp.tile(jnp.stack([img_w, img_h, img_w, img_h], axis=1), (1, 4))
    scores_p, labels_p, boxes_p = _post_process_sc(prob, boxes_flat, scale)
    scores = scores_p[:, :K]
    labels = labels_p[:, :K]
    boxes = boxes_p[:, :4 * K].reshape(B, K, 4)
    return scores, labels, boxes, image_names, target_sizes


# speedup vs baseline: 1.5579x; 1.5579x over previous
import jax
import jax.numpy as jnp
from jax import lax
from jax.experimental import pallas as pl
from jax.experimental.pallas import tpu as pltpu
from jax.experimental.pallas import tpu_sc as plsc

B = 64
Q = 900
C = 91
K = 300
QC = Q * C
L = 16
NV = (QC + L - 1) // L
QCP = NV * L
NC, NS = 2, 16
NW = NC * NS
BPW = B // NW
SEL = 320
KP = 304
HB = 1024
RS = HB * L
CAP = 2560
IDX_PAD = 0x7FFFFF00


def _take(v, idx):
    return lax.gather(
        v, idx[:, None],
        lax.GatherDimensionNumbers(
            offset_dims=(), collapsed_slice_dims=(0,), start_index_map=(0,)),
        (1,), mode=lax.GatherScatterMode.PROMISE_IN_BOUNDS)


def _splat(x, dtype=jnp.int32):
    return jnp.broadcast_to(jnp.asarray(x, dtype), (L,))


def _sc_body(prob_hbm, boxes_hbm, scale_hbm,
             scores_hbm, labels_hbm, boxes_out_hbm,
             p_v, boxes_v, scale_v, hist, hist2, sel_val, sel_idx,
             cand_val, cand_idx, oscore, olabel, oboxid, oboxes):
    cid = lax.axis_index("c")
    sid = lax.axis_index("s")
    wid = sid * NC + cid

    lanes = lax.iota(jnp.int32, L)
    ones = jnp.ones((L,), jnp.int32)
    lane_h = lanes * HB
    lane_r = lanes * 32
    rep4 = lanes >> 2
    mod4 = lanes & 3
    idx_cxy = (rep4 << 2) + (lanes & 1)
    idx_wh = idx_cxy + 2
    coef = jnp.where((lanes & 2) == 0, -0.5, 0.5).astype(jnp.float32)

    def load_p(i):
        v = p_v[pl.ds(i * L, L)]
        u = lax.bitcast_convert_type(v, jnp.int32)
        return v, u

    def in_group_pick(acc, needg):
        rev = lax.rev(acc, (0,))
        cs = plsc.cumsum(rev)
        m = cs >= _splat(needg)
        k = jnp.broadcast_to(plsc.all_reduce_ffs(m), (L,))
        sel = lanes == k
        k_s = jnp.sum(jnp.where(sel, lanes, 0))
        abv = jnp.sum(jnp.where(sel, cs - rev, 0))
        return k_s, abv

    def scan1024(need):
        @plsc.parallel_loop(0, HB, unroll=8)
        def _(t):
            hist[pl.ds(t * L, L)] = (hist[pl.ds(t * L, L)]
                                     + hist2[pl.ds(t * L, L)])

        @plsc.parallel_loop(
            0, 64, unroll=1,
            carry=(jnp.int32(0), jnp.int32(0), jnp.int32(0),
                   jnp.bool_(False)))
        def g_scan(t, carry):
            cum, gsel, above, found = carry
            g = 63 - t
            acc = hist[pl.ds(16 * g, L)]
            for l in range(1, L):
                acc = acc + hist[pl.ds(l * HB + 16 * g, L)]
            hist2[pl.ds(16 * g, L)] = acc
            tg = jnp.sum(acc)
            hit = jnp.logical_and(jnp.logical_not(found), cum + tg >= need)
            gsel = jnp.where(hit, g, gsel)
            above = jnp.where(hit, cum, above)
            found = jnp.logical_or(found, hit)
            return cum + tg, gsel, above, found
        _, gsel, above, _ = g_scan
        acc = hist2[pl.ds(16 * gsel, L)]
        k_s, abv_g = in_group_pick(acc, need - above)
        bucket = 16 * gsel + 15 - k_s
        above = above + abv_g
        cnt = jnp.sum(jnp.where(lanes == jnp.broadcast_to(k_s, (L,)),
                                lax.rev(acc, (0,)), 0))
        return bucket, above, cnt

    def scan32(need):
        t0 = hist[pl.ds(0, L)]
        t1 = hist[pl.ds(16, L)]
        for l in range(1, L):
            t0 = t0 + hist[pl.ds(l * 32, L)]
            t1 = t1 + hist[pl.ds(l * 32 + 16, L)]
        c1 = jnp.sum(t1)
        in_hi = need <= c1
        acc = jnp.where(_splat(in_hi, jnp.bool_), t1, t0)
        needg = jnp.where(in_hi, need, need - c1)
        k_s, abv_g = in_group_pick(acc, needg)
        bucket = jnp.where(in_hi, 16, 0) + 15 - k_s
        above = jnp.where(in_hi, jnp.int32(0), c1) + abv_g
        return bucket, above

    def zero_hist(n_vregs, both=False):
        @plsc.parallel_loop(0, n_vregs, unroll=8)
        def _(t):
            hist[pl.ds(t * L, L)] = jnp.zeros((L,), jnp.int32)
            if both:
                hist2[pl.ds(t * L, L)] = jnp.zeros((L,), jnp.int32)

    def append(buf_v, buf_i, off, m, v, gidx):
        mi = m.astype(jnp.int32)
        pos = off + plsc.cumsum(mi) - mi
        plsc.store_scatter(buf_v, [pos], v, mask=m)
        plsc.store_scatter(buf_i, [pos], gidx, mask=m)
        return off + plsc.all_reduce_population_count(m)

    def process(b):
        pltpu.sync_copy(prob_hbm.at[b], p_v)
        pltpu.sync_copy(boxes_hbm.at[b], boxes_v)
        pltpu.sync_copy(scale_hbm.at[b], scale_v)

        tail = p_v[pl.ds(QCP - L, L)]
        p_v[pl.ds(QCP - L, L)] = jnp.where(lanes < (L - (QCP - QC)),
                                           tail, 0.0)

        zero_hist(HB, both=True)

        def hist1_one(i, hb):
            v, u = load_p(i)
            d = lax.shift_right_logical(u, 20)
            plsc.addupdate_scatter(hb, [lane_h + d], ones)

        def hist1(g, _):
            for t in range(4):
                hist1_one(g * 4 + t, hist if t % 2 == 0 else hist2)
            return 0
        lax.fori_loop(0, NV // 4, hist1, 0)
        for i in range(NV - NV % 4, NV):
            hist1_one(jnp.int32(i), hist if i % 2 == 0 else hist2)
        b1, a1, cnt_b1 = scan1024(jnp.int32(K))
        need1 = jnp.int32(K) - a1

        @plsc.parallel_loop(0, SEL // L, unroll=4)
        def _(j):
            sel_val[pl.ds(j * L, L)] = jnp.full((L,), -1.0, jnp.float32)
            sel_idx[pl.ds(j * L, L)] = jnp.full((L,), IDX_PAD, jnp.int32)

        @plsc.parallel_loop(0, KP // L, unroll=4)
        def _(j):
            oboxid[pl.ds(j * L, L)] = jnp.zeros((L,), jnp.int32)

        def rounds_5bit(load_fn, nv, pfx0, need0):
            pfx, nd = pfx0, need0
            for shift in (15, 10, 5, 0):
                zero_hist(32)

                def hbody(i, _, shift=shift, pfx=pfx):
                    v, u, valid = load_fn(i)
                    m = lax.shift_right_logical(u, shift + 5) == _splat(pfx)
                    if valid is not None:
                        m = jnp.logical_and(valid, m)
                    d = lax.shift_right_logical(u, shift) & 31
                    plsc.addupdate_scatter(hist, [lane_r + d], ones, mask=m)
                    return 0
                lax.fori_loop(0, nv, hbody, 0)
                dsel, abv = scan32(nd)
                pfx = (pfx << 5) | dsel
                nd = nd - abv
            return pfx, nd

        def fast_path():
            @plsc.parallel_loop(0, NV, unroll=4, carry=(_splat(0), _splat(0)))
            def pass2(i, carry):
                off_s, off_c = carry
                v, u = load_p(i)
                gidx = _splat(i * L) + lanes
                d = lax.shift_right_logical(u, 20)
                off_s = append(sel_val, sel_idx, off_s, d > b1, v, gidx)
                off_c = append(cand_val, cand_idx, off_c, d == b1, v, gidx)
                return off_s, off_c

            ncv = lax.div(cnt_b1 + (L - 1), jnp.int32(L))

            def load_c(i):
                v = cand_val[pl.ds(i * L, L)]
                u = lax.bitcast_convert_type(v, jnp.int32)
                valid = (_splat(i * L) + lanes) < _splat(cnt_b1)
                return v, u, valid

            t_bits, need = rounds_5bit(load_c, ncv, b1, need1)

            @plsc.parallel_loop(0, ncv, unroll=4,
                                carry=(_splat(a1), _splat(0)))
            def _(i, carry):
                off, taken = carry
                v, u, valid = load_c(i)
                gidx = cand_idx[pl.ds(i * L, L)]
                m_gt = jnp.logical_and(u > t_bits, valid)
                m_eq = jnp.logical_and(
                    jnp.logical_and(u == t_bits, valid),
                    taken < _splat(need))
                off = append(sel_val, sel_idx,
                             off, jnp.logical_or(m_gt, m_eq), v, gidx)
                taken = taken + plsc.all_reduce_population_count(m_eq)
                return off, taken
            return jnp.int32(0)

        def slow_path():
            def load_f(i):
                v, u = load_p(i)
                return v, u, None

            t_bits, need = rounds_5bit(load_f, jnp.int32(NV), b1, need1)

            @plsc.parallel_loop(0, NV, unroll=4,
                                carry=(_splat(0), _splat(0)))
            def _(i, carry):
                off, taken = carry
                v, u = load_p(i)
                gidx = _splat(i * L) + lanes
                m_gt = u > t_bits
                m_eq = jnp.logical_and(u == t_bits, taken < _splat(need))
                off = append(sel_val, sel_idx,
                             off, jnp.logical_or(m_gt, m_eq), v, gidx)
                taken = taken + plsc.all_reduce_population_count(m_eq)
                return off, taken
            return jnp.int32(0)

        lax.cond(cnt_b1 <= CAP, fast_path, slow_path)

        @plsc.parallel_loop(0, SEL, unroll=4)
        def _(i):
            base = i - (i & (L - 1))
            lane = _splat(i & (L - 1))
            vi = _take(sel_val[pl.ds(base, L)], lane)
            ii = _take(sel_idx[pl.ds(base, L)], lane)
            i_s = _splat(i)
            rank = jnp.zeros((L,), jnp.int32)
            for j in range(SEL // L):
                va = sel_val[pl.ds(j * L, L)]
                pos = _splat(j * L) + lanes
                m = jnp.logical_or(
                    va > vi,
                    jnp.logical_and(va == vi, pos < i_s))
                rank = rank + plsc.all_reduce_population_count(m)
            mw = jnp.logical_and(lanes == 0, rank < K)
            plsc.store_scatter(oscore, [rank], vi, mask=mw)
            plsc.store_scatter(olabel, [rank], lax.rem(ii, _splat(C)),
                               mask=mw)
            plsc.store_scatter(oboxid, [rank], lax.div(ii, _splat(C)),
                               mask=mw)

        scale = scale_v[:]
        for vv in range(KP // 4):
            base = (4 * vv) & ~(L - 1)
            rows = _take(oboxid[pl.ds(base, L)], _splat(4 * vv - base) + rep4)
            bx = plsc.load_gather(boxes_v, [rows * 4 + mod4])
            out = (_take(bx, idx_cxy) + coef * _take(bx, idx_wh)) * scale
            oboxes[pl.ds(16 * vv, L)] = out

        pltpu.sync_copy(oscore, scores_hbm.at[b])
        pltpu.sync_copy(olabel, labels_hbm.at[b])
        pltpu.sync_copy(oboxes, boxes_out_hbm.at[b])

    for k in range(BPW):
        process(wid + NW * k)


@jax.jit
def _post_process_sc(prob, boxes_flat, scale):
    mesh = plsc.VectorSubcoreMesh(core_axis_name="c", subcore_axis_name="s",
                                  num_cores=NC, num_subcores=NS)
    fn = pl.kernel(
        _sc_body,
        out_type=[
            jax.ShapeDtypeStruct((B, KP), jnp.float32),
            jax.ShapeDtypeStruct((B, KP), jnp.int32),
            jax.ShapeDtypeStruct((B, 4 * KP), jnp.float32),
        ],
        mesh=mesh,
        compiler_params=pltpu.CompilerParams(needs_layout_passes=False,
                                             use_tc_tiling_on_sc=True),
        scratch_types=[
            pltpu.VMEM((QCP,), jnp.float32),
            pltpu.VMEM((4 * Q,), jnp.float32),
            pltpu.VMEM((L,), jnp.float32),
            pltpu.VMEM((RS,), jnp.int32),
            pltpu.VMEM((RS,), jnp.int32),
            pltpu.VMEM((SEL,), jnp.float32),
            pltpu.VMEM((SEL,), jnp.int32),
            pltpu.VMEM((CAP + L,), jnp.float32),
            pltpu.VMEM((CAP + L,), jnp.int32),
            pltpu.VMEM((KP,), jnp.float32),
            pltpu.VMEM((KP,), jnp.int32),
            pltpu.VMEM((KP,), jnp.int32),
            pltpu.VMEM((4 * KP,), jnp.float32),
        ],
    )
    return fn(prob, boxes_flat, scale)


def kernel(outputs_pred_logits, outputs_pred_boxes, target_sizes, image_names):
    prob = jax.nn.sigmoid(outputs_pred_logits).reshape(B, QC)
    prob = jnp.pad(prob, ((0, 0), (0, QCP - QC)), constant_values=-1.0)
    boxes_flat = outputs_pred_boxes.reshape(B, 4 * Q)
    img_h = target_sizes[:, 0].astype(jnp.float32)
    img_w = target_sizes[:, 1].astype(jnp.float32)
    scale = jnp.tile(jnp.stack([img_w, img_h, img_w, img_h], axis=1), (1, 4))
    scores_p, labels_p, boxes_p = _post_process_sc(prob, boxes_flat, scale)
    scores = scores_p[:, :K]
    labels = labels_p[:, :K]
    boxes = boxes_p[:, :4 * K].reshape(B, K, 4)
    return scores, labels, boxes, image_names, target_sizes


# speedup vs baseline: 1.6262x; 1.0438x over previous
import jax
import jax.numpy as jnp
from jax import lax
from jax.experimental import pallas as pl
from jax.experimental.pallas import tpu as pltpu
from jax.experimental.pallas import tpu_sc as plsc

B = 64
Q = 900
C = 91
K = 300
QC = Q * C
L = 16
NV = (QC + L - 1) // L
QCP = NV * L
NC, NS = 2, 16
NW = NC * NS
BPW = B // NW
SEL = 320
KP = 304
HB = 1024
RS = HB * L
CAP = 2560
IDX_PAD = 0x7FFFFF00


def _take(v, idx):
    return lax.gather(
        v, idx[:, None],
        lax.GatherDimensionNumbers(
            offset_dims=(), collapsed_slice_dims=(0,), start_index_map=(0,)),
        (1,), mode=lax.GatherScatterMode.PROMISE_IN_BOUNDS)


def _splat(x, dtype=jnp.int32):
    return jnp.broadcast_to(jnp.asarray(x, dtype), (L,))


def _sc_body(prob_hbm, boxes_hbm, scale_hbm,
             scores_hbm, labels_hbm, boxes_out_hbm,
             p_v, boxes_v, scale_v, hist, hist2, sel_val, sel_idx,
             cand_val, cand_idx, oscore, olabel, oboxid, oboxes):
    cid = lax.axis_index("c")
    sid = lax.axis_index("s")
    wid = sid * NC + cid

    lanes = lax.iota(jnp.int32, L)
    ones = jnp.ones((L,), jnp.int32)
    lane_h = lanes * HB
    lane_r = lanes * 32
    rep4 = lanes >> 2
    mod4 = lanes & 3
    idx_cxy = (rep4 << 2) + (lanes & 1)
    idx_wh = idx_cxy + 2
    coef = jnp.where((lanes & 2) == 0, -0.5, 0.5).astype(jnp.float32)

    def load_p(i):
        v = p_v[pl.ds(i * L, L)]
        u = lax.bitcast_convert_type(v, jnp.int32)
        return v, u

    def in_group_pick(acc, needg):
        rev = lax.rev(acc, (0,))
        cs = plsc.cumsum(rev)
        m = cs >= _splat(needg)
        k = jnp.broadcast_to(plsc.all_reduce_ffs(m), (L,))
        sel = lanes == k
        k_s = jnp.sum(jnp.where(sel, lanes, 0))
        abv = jnp.sum(jnp.where(sel, cs - rev, 0))
        return k_s, abv

    def scan1024(need):
        @plsc.parallel_loop(0, HB, unroll=8)
        def _(t):
            hist[pl.ds(t * L, L)] = (hist[pl.ds(t * L, L)]
                                     + hist2[pl.ds(t * L, L)])

        @plsc.parallel_loop(
            0, 64, unroll=1,
            carry=(jnp.int32(0), jnp.int32(0), jnp.int32(0),
                   jnp.bool_(False)))
        def g_scan(t, carry):
            cum, gsel, above, found = carry
            g = 63 - t
            acc = hist[pl.ds(16 * g, L)]
            for l in range(1, L):
                acc = acc + hist[pl.ds(l * HB + 16 * g, L)]
            hist2[pl.ds(16 * g, L)] = acc
            tg = jnp.sum(acc)
            hit = jnp.logical_and(jnp.logical_not(found), cum + tg >= need)
            gsel = jnp.where(hit, g, gsel)
            above = jnp.where(hit, cum, above)
            found = jnp.logical_or(found, hit)
            return cum + tg, gsel, above, found
        _, gsel, above, _ = g_scan
        acc = hist2[pl.ds(16 * gsel, L)]
        k_s, abv_g = in_group_pick(acc, need - above)
        bucket = 16 * gsel + 15 - k_s
        above = above + abv_g
        cnt = jnp.sum(jnp.where(lanes == jnp.broadcast_to(k_s, (L,)),
                                lax.rev(acc, (0,)), 0))
        return bucket, above, cnt

    def scan32(need):
        t0 = hist[pl.ds(0, L)]
        t1 = hist[pl.ds(16, L)]
        for l in range(1, L):
            t0 = t0 + hist[pl.ds(l * 32, L)]
            t1 = t1 + hist[pl.ds(l * 32 + 16, L)]
        c1 = jnp.sum(t1)
        in_hi = need <= c1
        acc = jnp.where(_splat(in_hi, jnp.bool_), t1, t0)
        needg = jnp.where(in_hi, need, need - c1)
        k_s, abv_g = in_group_pick(acc, needg)
        bucket = jnp.where(in_hi, 16, 0) + 15 - k_s
        above = jnp.where(in_hi, jnp.int32(0), c1) + abv_g
        return bucket, above

    def zero_hist(n_vregs, both=False):
        @plsc.parallel_loop(0, n_vregs, unroll=8)
        def _(t):
            hist[pl.ds(t * L, L)] = jnp.zeros((L,), jnp.int32)
            if both:
                hist2[pl.ds(t * L, L)] = jnp.zeros((L,), jnp.int32)

    def append(buf_v, buf_i, off, m, v, gidx):
        mi = m.astype(jnp.int32)
        pos = off + plsc.cumsum(mi) - mi
        plsc.store_scatter(buf_v, [pos], v, mask=m)
        plsc.store_scatter(buf_i, [pos], gidx, mask=m)
        return off + plsc.all_reduce_population_count(m)

    def process(b):
        pltpu.sync_copy(prob_hbm.at[b], p_v)
        pltpu.sync_copy(boxes_hbm.at[b], boxes_v)
        pltpu.sync_copy(scale_hbm.at[b], scale_v)

        tail = p_v[pl.ds(QCP - L, L)]
        p_v[pl.ds(QCP - L, L)] = jnp.where(lanes < (L - (QCP - QC)),
                                           tail, 0.0)

        zero_hist(HB, both=True)

        def hist1_one(i, hb):
            v, u = load_p(i)
            d = lax.shift_right_logical(u, 20)
            plsc.addupdate_scatter(hb, [lane_h + d], ones)

        def hist1(g, _):
            for t in range(4):
                hist1_one(g * 4 + t, hist if t % 2 == 0 else hist2)
            return 0
        lax.fori_loop(0, NV // 4, hist1, 0)
        for i in range(NV - NV % 4, NV):
            hist1_one(jnp.int32(i), hist if i % 2 == 0 else hist2)
        b1, a1, cnt_b1 = scan1024(jnp.int32(K))
        need1 = jnp.int32(K) - a1

        @plsc.parallel_loop(0, SEL // L, unroll=4)
        def _(j):
            sel_val[pl.ds(j * L, L)] = jnp.full((L,), -1.0, jnp.float32)
            sel_idx[pl.ds(j * L, L)] = jnp.full((L,), IDX_PAD, jnp.int32)

        @plsc.parallel_loop(0, KP // L, unroll=4)
        def _(j):
            oboxid[pl.ds(j * L, L)] = jnp.zeros((L,), jnp.int32)

        def rounds_5bit(load_fn, nv, pfx0, need0):
            pfx, nd = pfx0, need0
            for shift in (15, 10, 5, 0):
                zero_hist(32)

                def hbody(i, _, shift=shift, pfx=pfx):
                    v, u, valid = load_fn(i)
                    m = lax.shift_right_logical(u, shift + 5) == _splat(pfx)
                    if valid is not None:
                        m = jnp.logical_and(valid, m)
                    d = lax.shift_right_logical(u, shift) & 31
                    plsc.addupdate_scatter(hist, [lane_r + d], ones, mask=m)
                    return 0
                lax.fori_loop(0, nv, hbody, 0)
                dsel, abv = scan32(nd)
                pfx = (pfx << 5) | dsel
                nd = nd - abv
            return pfx, nd

        def fast_path():
            @plsc.parallel_loop(0, NV, unroll=4, carry=_splat(0))
            def pass2(i, off_c):
                v, u = load_p(i)
                gidx = _splat(i * L) + lanes
                d = lax.shift_right_logical(u, 20)
                return append(cand_val, cand_idx, off_c, d >= b1, v, gidx)

            n_cand = cnt_b1 + a1
            ncv = lax.div(n_cand + (L - 1), jnp.int32(L))

            def load_c(i):
                v = cand_val[pl.ds(i * L, L)]
                u = lax.bitcast_convert_type(v, jnp.int32)
                valid = (_splat(i * L) + lanes) < _splat(n_cand)
                return v, u, valid

            t_bits, need = rounds_5bit(load_c, ncv, b1, need1)

            @plsc.parallel_loop(0, ncv, unroll=4,
                                carry=(_splat(0), _splat(0)))
            def _(i, carry):
                off, taken = carry
                v, u, valid = load_c(i)
                gidx = cand_idx[pl.ds(i * L, L)]
                m_gt = jnp.logical_and(u > t_bits, valid)
                m_eq = jnp.logical_and(
                    jnp.logical_and(u == t_bits, valid),
                    taken < _splat(need))
                off = append(sel_val, sel_idx,
                             off, jnp.logical_or(m_gt, m_eq), v, gidx)
                taken = taken + plsc.all_reduce_population_count(m_eq)
                return off, taken
            return jnp.int32(0)

        def slow_path():
            def load_f(i):
                v, u = load_p(i)
                return v, u, None

            t_bits, need = rounds_5bit(load_f, jnp.int32(NV), b1, need1)

            @plsc.parallel_loop(0, NV, unroll=4,
                                carry=(_splat(0), _splat(0)))
            def _(i, carry):
                off, taken = carry
                v, u = load_p(i)
                gidx = _splat(i * L) + lanes
                m_gt = u > t_bits
                m_eq = jnp.logical_and(u == t_bits, taken < _splat(need))
                off = append(sel_val, sel_idx,
                             off, jnp.logical_or(m_gt, m_eq), v, gidx)
                taken = taken + plsc.all_reduce_population_count(m_eq)
                return off, taken
            return jnp.int32(0)

        lax.cond(cnt_b1 + a1 <= CAP, fast_path, slow_path)

        @plsc.parallel_loop(0, SEL, unroll=8)
        def _(i):
            base = i - (i & (L - 1))
            lane = _splat(i & (L - 1))
            vi = _take(sel_val[pl.ds(base, L)], lane)
            ii = _take(sel_idx[pl.ds(base, L)], lane)
            i_s = _splat(i)
            rank = jnp.zeros((L,), jnp.int32)
            for j in range(SEL // L):
                va = sel_val[pl.ds(j * L, L)]
                pos = _splat(j * L) + lanes
                m = jnp.logical_or(
                    va > vi,
                    jnp.logical_and(va == vi, pos < i_s))
                rank = rank + plsc.all_reduce_population_count(m)
            mw = jnp.logical_and(lanes == 0, rank < K)
            plsc.store_scatter(oscore, [rank], vi, mask=mw)
            plsc.store_scatter(olabel, [rank], lax.rem(ii, _splat(C)),
                               mask=mw)
            plsc.store_scatter(oboxid, [rank], lax.div(ii, _splat(C)),
                               mask=mw)

        scale = scale_v[:]
        for vv in range(KP // 4):
            base = (4 * vv) & ~(L - 1)
            rows = _take(oboxid[pl.ds(base, L)], _splat(4 * vv - base) + rep4)
            bx = plsc.load_gather(boxes_v, [rows * 4 + mod4])
            out = (_take(bx, idx_cxy) + coef * _take(bx, idx_wh)) * scale
            oboxes[pl.ds(16 * vv, L)] = out

        pltpu.sync_copy(oscore, scores_hbm.at[b])
        pltpu.sync_copy(olabel, labels_hbm.at[b])
        pltpu.sync_copy(oboxes, boxes_out_hbm.at[b])

    for k in range(BPW):
        process(wid + NW * k)


@jax.jit
def _post_process_sc(prob, boxes_flat, scale):
    mesh = plsc.VectorSubcoreMesh(core_axis_name="c", subcore_axis_name="s",
                                  num_cores=NC, num_subcores=NS)
    fn = pl.kernel(
        _sc_body,
        out_type=[
            jax.ShapeDtypeStruct((B, KP), jnp.float32),
            jax.ShapeDtypeStruct((B, KP), jnp.int32),
            jax.ShapeDtypeStruct((B, 4 * KP), jnp.float32),
        ],
        mesh=mesh,
        compiler_params=pltpu.CompilerParams(needs_layout_passes=False,
                                             use_tc_tiling_on_sc=True),
        scratch_types=[
            pltpu.VMEM((QCP,), jnp.float32),
            pltpu.VMEM((4 * Q,), jnp.float32),
            pltpu.VMEM((L,), jnp.float32),
            pltpu.VMEM((RS,), jnp.int32),
            pltpu.VMEM((RS,), jnp.int32),
            pltpu.VMEM((SEL,), jnp.float32),
            pltpu.VMEM((SEL,), jnp.int32),
            pltpu.VMEM((CAP + L,), jnp.float32),
            pltpu.VMEM((CAP + L,), jnp.int32),
            pltpu.VMEM((KP,), jnp.float32),
            pltpu.VMEM((KP,), jnp.int32),
            pltpu.VMEM((KP,), jnp.int32),
            pltpu.VMEM((4 * KP,), jnp.float32),
        ],
    )
    return fn(prob, boxes_flat, scale)


def kernel(outputs_pred_logits, outputs_pred_boxes, target_sizes, image_names):
    prob = jax.nn.sigmoid(outputs_pred_logits).reshape(B, QC)
    prob = jnp.pad(prob, ((0, 0), (0, QCP - QC)), constant_values=-1.0)
    boxes_flat = outputs_pred_boxes.reshape(B, 4 * Q)
    img_h = target_sizes[:, 0].astype(jnp.float32)
    img_w = target_sizes[:, 1].astype(jnp.float32)
    scale = jnp.tile(jnp.stack([img_w, img_h, img_w, img_h], axis=1), (1, 4))
    scores_p, labels_p, boxes_p = _post_process_sc(prob, boxes_flat, scale)
    scores = scores_p[:, :K]
    labels = labels_p[:, :K]
    boxes = boxes_p[:, :4 * K].reshape(B, K, 4)
    return scores, labels, boxes, image_names, target_sizes
